# Initial kernel scaffold; baseline (speedup 1.0000x reference)
import jax
import jax.numpy as jnp
from jax.experimental import pallas as pl


def kernel(indices):
    raise NotImplementedError("write your pallas kernel here")


# speedup vs baseline: 1.7090x; 1.7090x over previous
import functools

import jax
import jax.numpy as jnp
from jax import lax
from jax.experimental import pallas as pl
from jax.experimental.pallas import tpu as pltpu
from jax.experimental.pallas import tpu_sc as plsc

NBINS = 8192
EPS = 1e-08
NC = 2
NS = 16
L = 16
NW = NC * NS
NCOPY = 8
CHUNK = 16384
UNROLL = 8


def _sc_histogram(flat):
    e = flat.shape[0]
    e_per = e // NW
    nchunk = e_per // CHUNK
    assert e_per % CHUNK == 0 and nchunk % 2 == 0

    mesh = plsc.VectorSubcoreMesh(core_axis_name="c", subcore_axis_name="s")

    @functools.partial(
        pl.kernel,
        out_type=jax.ShapeDtypeStruct((NW, NBINS), jnp.float32),
        mesh=mesh,
        scratch_types=[
            pltpu.VMEM((2, CHUNK), jnp.int32),
            pltpu.VMEM((NCOPY * NBINS,), jnp.float32),
            pltpu.SemaphoreType.DMA,
            pltpu.SemaphoreType.DMA,
        ],
        compiler_params=pltpu.CompilerParams(needs_layout_passes=False),
    )
    def hist_kernel(idx_hbm, out_hbm, inbuf, hist, sem0, sem1):
        wid = lax.axis_index("c") * NS + lax.axis_index("s")
        base = wid * e_per
        sems = (sem0, sem1)

        lane = lax.iota(jnp.int32, L)
        laneoff = (lane & (NCOPY - 1)) * NBINS
        ones = jnp.full((L,), 1.0, jnp.float32)
        mlo = lane < (L // 2)
        mhi = lane >= (L // 2)

        def zero_body(i, c):
            hist[pl.ds(i * L, L)] = jnp.zeros((L,), jnp.float32)
            return c
        lax.fori_loop(0, NCOPY * NBINS // L, zero_body, 0)

        for b in range(2):
            pltpu.async_copy(
                idx_hbm.at[pl.ds(base + b * CHUNK, CHUNK)], inbuf.at[b], sems[b]
            )

        def consume(b):
            def body(j, c):
                for u in range(UNROLL):
                    idx = inbuf[b, pl.ds(j * (L * UNROLL) + u * L, L)]
                    addr = idx + laneoff
                    plsc.addupdate_scatter(hist, [addr], ones, mask=mlo)
                    plsc.addupdate_scatter(hist, [addr], ones, mask=mhi)
                return c
            lax.fori_loop(0, CHUNK // (L * UNROLL), body, 0)

        def pair_body(p, c):
            for b in range(2):
                cur = 2 * p + b
                pltpu.make_async_copy(
                    idx_hbm.at[pl.ds(base + cur * CHUNK, CHUNK)],
                    inbuf.at[b],
                    sems[b],
                ).wait()
                consume(b)
                pltpu.async_copy(
                    idx_hbm.at[pl.ds(base + (cur + 2) * CHUNK, CHUNK)],
                    inbuf.at[b],
                    sems[b],
                )
            return c
        lax.fori_loop(0, nchunk // 2 - 1, pair_body, 0)

        for b in range(2):
            cur = nchunk - 2 + b
            pltpu.make_async_copy(
                idx_hbm.at[pl.ds(base + cur * CHUNK, CHUNK)],
                inbuf.at[b],
                sems[b],
            ).wait()
            consume(b)

        def reduce_body(k, c):
            acc = hist[pl.ds(k * L, L)]
            for j in range(1, NCOPY):
                acc = acc + hist[pl.ds(j * NBINS + k * L, L)]
            hist[pl.ds(k * L, L)] = acc
            return c
        lax.fori_loop(0, NBINS // L, reduce_body, 0)

        pltpu.sync_copy(hist.at[pl.ds(0, NBINS)], out_hbm.at[wid])

    return hist_kernel(flat)


def _tc_perplexity(partials):
    def body(p_ref, o_ref):
        x = p_ref[...]
        counts = jnp.sum(x, axis=0)
        total = jnp.sum(counts)
        probs = counts / total
        entropy = -jnp.sum(probs * jnp.log(probs + EPS))
        o_ref[...] = jnp.exp(entropy)[None, None]

    return pl.pallas_call(
        body,
        out_shape=jax.ShapeDtypeStruct((1, 1), jnp.float32),
    )(partials)


def kernel(indices):
    flat = indices.reshape(-1)
    partials = _sc_histogram(flat)
    out = _tc_perplexity(partials.reshape(NW, NBINS // 128, 128))
    return out[0, 0]


# speedup vs baseline: 2.0786x; 1.2163x over previous
import functools

import jax
import jax.numpy as jnp
from jax import lax
from jax.experimental import pallas as pl
from jax.experimental.pallas import tpu as pltpu
from jax.experimental.pallas import tpu_sc as plsc

NBINS = 8192
EPS = 1e-08
NC = 2
NS = 16
L = 16
NW = NC * NS
NCOPY = 8
CHUNK = 16384
UNROLL = 8


def _sc_histogram(indices):
    nrow, ncol = indices.shape
    rows_per_chunk = CHUNK // ncol
    rows_per_tile = nrow // NW
    nchunk = rows_per_tile // rows_per_chunk
    assert CHUNK % ncol == 0 and rows_per_tile % rows_per_chunk == 0
    assert nchunk % 2 == 0

    mesh = plsc.VectorSubcoreMesh(core_axis_name="c", subcore_axis_name="s")

    @functools.partial(
        pl.kernel,
        out_type=jax.ShapeDtypeStruct((NW, NBINS), jnp.float32),
        mesh=mesh,
        scratch_types=[
            pltpu.VMEM((2, rows_per_chunk, ncol), jnp.int32),
            pltpu.VMEM((NCOPY * NBINS,), jnp.float32),
            pltpu.SemaphoreType.DMA,
            pltpu.SemaphoreType.DMA,
        ],
        compiler_params=pltpu.CompilerParams(needs_layout_passes=False),
    )
    def hist_kernel(idx_hbm, out_hbm, inbuf, hist, sem0, sem1):
        wid = lax.axis_index("c") * NS + lax.axis_index("s")
        row_base = wid * rows_per_tile
        sems = (sem0, sem1)

        lane = lax.iota(jnp.int32, L)
        laneoff = (lane & (NCOPY - 1)) * NBINS
        ones = jnp.full((L,), 1.0, jnp.float32)
        mlo = lane < (L // 2)
        mhi = lane >= (L // 2)

        def zero_body(i, c):
            hist[pl.ds(i * L, L)] = jnp.zeros((L,), jnp.float32)
            return c
        lax.fori_loop(0, NCOPY * NBINS // L, zero_body, 0)

        def chunk_src(c):
            return idx_hbm.at[pl.ds(row_base + c * rows_per_chunk, rows_per_chunk)]

        for b in range(2):
            pltpu.async_copy(chunk_src(b), inbuf.at[b], sems[b])

        def consume(b):
            for r in range(rows_per_chunk):
                def body(j, c):
                    for u in range(UNROLL):
                        idx = inbuf[b, r, pl.ds(j * (L * UNROLL) + u * L, L)]
                        addr = idx + laneoff
                        plsc.addupdate_scatter(hist, [addr], ones, mask=mlo)
                        plsc.addupdate_scatter(hist, [addr], ones, mask=mhi)
                    return c
                lax.fori_loop(0, ncol // (L * UNROLL), body, 0)

        def pair_body(p, c):
            for b in range(2):
                cur = 2 * p + b
                pltpu.make_async_copy(chunk_src(cur), inbuf.at[b], sems[b]).wait()
                consume(b)
                pltpu.async_copy(chunk_src(cur + 2), inbuf.at[b], sems[b])
            return c
        lax.fori_loop(0, nchunk // 2 - 1, pair_body, 0)

        for b in range(2):
            cur = nchunk - 2 + b
            pltpu.make_async_copy(chunk_src(cur), inbuf.at[b], sems[b]).wait()
            consume(b)

        def reduce_body(k, c):
            acc = hist[pl.ds(k * L, L)]
            for j in range(1, NCOPY):
                acc = acc + hist[pl.ds(j * NBINS + k * L, L)]
            hist[pl.ds(k * L, L)] = acc
            return c
        lax.fori_loop(0, NBINS // L, reduce_body, 0)

        pltpu.sync_copy(hist.at[pl.ds(0, NBINS)], out_hbm.at[wid])

    return hist_kernel(indices)


def _tc_perplexity(partials):
    def body(p_ref, o_ref):
        x = p_ref[...]
        counts = jnp.sum(x, axis=0)
        total = jnp.sum(counts)
        probs = counts / total
        entropy = -jnp.sum(probs * jnp.log(probs + EPS))
        o_ref[...] = jnp.exp(entropy)[None, None]

    return pl.pallas_call(
        body,
        out_shape=jax.ShapeDtypeStruct((1, 1), jnp.float32),
    )(partials)


def kernel(indices):
    partials = _sc_histogram(indices)
    out = _tc_perplexity(partials.reshape(NW, NBINS // 128, 128))
    return out[0, 0]


# speedup vs baseline: 5.4200x; 2.6075x over previous
import functools

import jax
import jax.numpy as jnp
from jax import lax
from jax.experimental import pallas as pl
from jax.experimental.pallas import tpu as pltpu
from jax.experimental.pallas import tpu_sc as plsc

NBINS = 8192
EPS = 1e-08
NC = 2
NS = 16
L = 16
NW = NC * NS
NCOPY = 8
CHUNK = 16384
UNROLL = 8


def _sc_histogram(indices):
    nrow, ncol = indices.shape
    rows_per_chunk = CHUNK // ncol
    rows_per_tile = nrow // NW
    nchunk = rows_per_tile // rows_per_chunk
    assert CHUNK % ncol == 0 and rows_per_tile % rows_per_chunk == 0
    assert nchunk % 2 == 0

    mesh = plsc.VectorSubcoreMesh(core_axis_name="c", subcore_axis_name="s")

    @functools.partial(
        pl.kernel,
        out_type=jax.ShapeDtypeStruct((NW, NBINS), jnp.float32),
        mesh=mesh,
        scratch_types=[
            pltpu.VMEM((2, rows_per_chunk, ncol), jnp.int32),
            pltpu.VMEM((NCOPY * NBINS,), jnp.float32),
            pltpu.SemaphoreType.DMA,
            pltpu.SemaphoreType.DMA,
        ],
        compiler_params=pltpu.CompilerParams(needs_layout_passes=False),
    )
    def hist_kernel(idx_hbm, out_hbm, inbuf, hist, sem0, sem1):
        wid = lax.axis_index("c") * NS + lax.axis_index("s")
        row_base = wid * rows_per_tile
        sems = (sem0, sem1)

        lane = lax.iota(jnp.int32, L)
        laneoff = (lane & (NCOPY - 1)) * NBINS
        ones = jnp.full((L,), 1.0, jnp.float32)
        mlo = lane < (L // 2)
        mhi = lane >= (L // 2)

        @plsc.parallel_loop(0, NCOPY * NBINS // L, unroll=4)
        def _zero(i):
            hist[pl.ds(i * L, L)] = jnp.zeros((L,), jnp.float32)

        def chunk_src(c):
            return idx_hbm.at[pl.ds(row_base + c * rows_per_chunk, rows_per_chunk)]

        for b in range(2):
            pltpu.async_copy(chunk_src(b), inbuf.at[b], sems[b])

        def consume(b):
            for r in range(rows_per_chunk):
                @plsc.parallel_loop(0, ncol // L, unroll=UNROLL)
                def _scatter(i):
                    idx = inbuf[b, r, pl.ds(i * L, L)]
                    addr = idx + laneoff
                    plsc.addupdate_scatter(hist, [addr], ones, mask=mlo)
                    plsc.addupdate_scatter(hist, [addr], ones, mask=mhi)

        def pair_body(p, c):
            for b in range(2):
                cur = 2 * p + b
                pltpu.make_async_copy(chunk_src(cur), inbuf.at[b], sems[b]).wait()
                consume(b)
                pltpu.async_copy(chunk_src(cur + 2), inbuf.at[b], sems[b])
            return c
        lax.fori_loop(0, nchunk // 2 - 1, pair_body, 0)

        for b in range(2):
            cur = nchunk - 2 + b
            pltpu.make_async_copy(chunk_src(cur), inbuf.at[b], sems[b]).wait()
            consume(b)

        @plsc.parallel_loop(0, NBINS // L, unroll=4)
        def _reduce(k):
            acc = hist[pl.ds(k * L, L)]
            for j in range(1, NCOPY):
                acc = acc + hist[pl.ds(j * NBINS + k * L, L)]
            hist[pl.ds(k * L, L)] = acc

        pltpu.sync_copy(hist.at[pl.ds(0, NBINS)], out_hbm.at[wid])

    return hist_kernel(indices)


def _tc_perplexity(partials):
    def body(p_ref, o_ref):
        x = p_ref[...]
        counts = jnp.sum(x, axis=0)
        total = jnp.sum(counts)
        probs = counts / total
        entropy = -jnp.sum(probs * jnp.log(probs + EPS))
        o_ref[...] = jnp.exp(entropy)[None, None]

    return pl.pallas_call(
        body,
        out_shape=jax.ShapeDtypeStruct((1, 1), jnp.float32),
    )(partials)


def kernel(indices):
    partials = _sc_histogram(indices)
    out = _tc_perplexity(partials.reshape(NW, NBINS // 128, 128))
    return out[0, 0]


# speedup vs baseline: 6.8990x; 1.2729x over previous
import functools

import jax
import jax.numpy as jnp
from jax import lax
from jax.experimental import pallas as pl
from jax.experimental.pallas import tpu as pltpu
from jax.experimental.pallas import tpu_sc as plsc

NBINS = 8192
EPS = 1e-08
NC = 2
NS = 16
L = 16
NW = NC * NS
NCOPY = 8
CHUNK = 16384
UNROLL = 8


def _sc_histogram(indices):
    nrow, ncol = indices.shape
    rows_per_chunk = CHUNK // ncol
    rows_per_tile = nrow // NW
    nchunk = rows_per_tile // rows_per_chunk
    assert CHUNK % ncol == 0 and rows_per_tile % rows_per_chunk == 0
    assert nchunk % 2 == 0

    mesh = plsc.VectorSubcoreMesh(core_axis_name="c", subcore_axis_name="s")

    @functools.partial(
        pl.kernel,
        out_type=jax.ShapeDtypeStruct((NW, NBINS), jnp.float32),
        mesh=mesh,
        scratch_types=[
            pltpu.VMEM((2, rows_per_chunk, ncol), jnp.int32),
            pltpu.VMEM((NBINS,), jnp.float32),
            pltpu.SemaphoreType.DMA,
            pltpu.SemaphoreType.DMA,
        ],
        compiler_params=pltpu.CompilerParams(needs_layout_passes=False),
    )
    def hist_kernel(idx_hbm, out_hbm, inbuf, hist, sem0, sem1):
        wid = lax.axis_index("c") * NS + lax.axis_index("s")
        row_base = wid * rows_per_tile
        sems = (sem0, sem1)

        ones = jnp.full((L,), 1.0, jnp.float32)

        @plsc.parallel_loop(0, NBINS // L, unroll=4)
        def _zero(i):
            hist[pl.ds(i * L, L)] = jnp.zeros((L,), jnp.float32)

        def chunk_src(c):
            return idx_hbm.at[pl.ds(row_base + c * rows_per_chunk, rows_per_chunk)]

        for b in range(2):
            pltpu.async_copy(chunk_src(b), inbuf.at[b], sems[b])

        def consume(b):
            for r in range(rows_per_chunk):
                @plsc.parallel_loop(0, ncol // L, unroll=UNROLL)
                def _scatter(i):
                    idx = inbuf[b, r, pl.ds(i * L, L)]
                    plsc.addupdate_scatter(hist, [idx], ones)

        def pair_body(p, c):
            for b in range(2):
                cur = 2 * p + b
                pltpu.make_async_copy(chunk_src(cur), inbuf.at[b], sems[b]).wait()
                consume(b)
                pltpu.async_copy(chunk_src(cur + 2), inbuf.at[b], sems[b])
            return c
        lax.fori_loop(0, nchunk // 2 - 1, pair_body, 0)

        for b in range(2):
            cur = nchunk - 2 + b
            pltpu.make_async_copy(chunk_src(cur), inbuf.at[b], sems[b]).wait()
            consume(b)

        pltpu.sync_copy(hist.at[pl.ds(0, NBINS)], out_hbm.at[wid])

    return hist_kernel(indices)


def _tc_perplexity(partials):
    def body(p_ref, o_ref):
        x = p_ref[...]
        counts = jnp.sum(x, axis=0)
        total = jnp.sum(counts)
        probs = counts / total
        entropy = -jnp.sum(probs * jnp.log(probs + EPS))
        o_ref[...] = jnp.exp(entropy)[None, None]

    return pl.pallas_call(
        body,
        out_shape=jax.ShapeDtypeStruct((1, 1), jnp.float32),
    )(partials)


def kernel(indices):
    partials = _sc_histogram(indices)
    out = _tc_perplexity(partials.reshape(NW, NBINS // 128, 128))
    return out[0, 0]
